# baseline (speedup 1.0000x reference)
import jax
import jax.numpy as jnp
from jax import lax
from jax.experimental import pallas as pl
from jax.experimental.pallas import tpu as pltpu
from jax.experimental.pallas import tpu_sc as plsc

B, SEQ, D = 4, 8192, 1024
NC, NS = 2, 16
NW = NC * NS
ROWS_PER_W = SEQ // NW
CHUNK = 8
NCHUNK = ROWS_PER_W // CHUNK
CH = CHUNK * D


def _sc_body(in_hbm, tab_hbm, out_hbm, tab_v, io_v):
    wid = lax.axis_index("s") * NC + lax.axis_index("c")
    base = wid * (ROWS_PER_W * D)

    def chunk_body(ci, carry):
        off = base + ci * CH
        pltpu.sync_copy(tab_hbm.at[pl.ds(off, CH)], tab_v)
        for b in range(B):
            pltpu.sync_copy(in_hbm.at[b, pl.ds(off, CH)], io_v.at[b])

        def vec_body(n, c2):
            voff = n * 16
            t = tab_v[pl.ds(voff, 16)]
            for b in range(B):
                io_v[b, pl.ds(voff, 16)] = io_v[b, pl.ds(voff, 16)] + t
            return c2

        lax.fori_loop(0, CH // 16, vec_body, 0, unroll=2)
        for b in range(B):
            pltpu.sync_copy(io_v.at[b], out_hbm.at[b, pl.ds(off, CH)])
        return carry

    lax.fori_loop(0, NCHUNK, chunk_body, 0)


def kernel(inputs, pos_table):
    in_flat = inputs.reshape(B, SEQ * D)
    tab_flat = pos_table.reshape(SEQ * D)
    mesh = plsc.VectorSubcoreMesh(core_axis_name="c", subcore_axis_name="s")
    out = pl.kernel(
        _sc_body,
        mesh=mesh,
        out_type=jax.ShapeDtypeStruct((B, SEQ * D), jnp.float32),
        scratch_types=[
            pltpu.VMEM((CH,), jnp.float32),
            pltpu.VMEM((B, CH), jnp.float32),
        ],
    )(in_flat, tab_flat)
    return out.reshape(B, SEQ, D)


# speedup vs baseline: 1.6862x; 1.6862x over previous
import jax
import jax.numpy as jnp
from jax import lax
from jax.experimental import pallas as pl
from jax.experimental.pallas import tpu as pltpu
from jax.experimental.pallas import tpu_sc as plsc

B, SEQ, D = 4, 8192, 1024
NC, NS = 2, 16
NW = NC * NS
ROWS_PER_W = SEQ // NW
CHUNK = 8
NCHUNK = ROWS_PER_W // CHUNK
NBUF = 2
LANES = 16
VECS_PER_ROW = D // LANES


def _sc_body(in_hbm, tab_hbm, out_hbm, tab_v, io_v, in_sems, out_sems):
    wid = lax.axis_index("s") * NC + lax.axis_index("c")
    row_base = wid * ROWS_PER_W

    def in_cps(ci, sl):
        r0 = row_base + ci * CHUNK
        return [
            pltpu.make_async_copy(
                tab_hbm.at[pl.ds(r0, CHUNK)], tab_v.at[sl], in_sems.at[sl]),
            pltpu.make_async_copy(
                in_hbm.at[:, pl.ds(r0, CHUNK), :], io_v.at[sl], in_sems.at[sl]),
        ]

    def out_cps(ci, sl):
        r0 = row_base + ci * CHUNK
        return [
            pltpu.make_async_copy(
                io_v.at[sl], out_hbm.at[:, pl.ds(r0, CHUNK), :], out_sems.at[sl]),
        ]

    def issue(cps):
        for c in cps:
            c.start()

    def drain(cps):
        for c in cps:
            c.wait()

    def compute(sl):
        def vec_body(n, c2):
            r = n // VECS_PER_ROW
            col = (n % VECS_PER_ROW) * LANES
            t = tab_v[sl, r, pl.ds(col, LANES)]
            for b in range(B):
                io_v[sl, b, r, pl.ds(col, LANES)] = (
                    io_v[sl, b, r, pl.ds(col, LANES)] + t)
            return c2

        lax.fori_loop(0, CHUNK * VECS_PER_ROW, vec_body, 0, unroll=4)

    issue(in_cps(0, 0))

    def chunk_step(ci, carry):
        sl = lax.rem(ci, NBUF)
        nsl = 1 - sl

        @pl.when(ci + 1 < NCHUNK)
        def _prefetch():
            @pl.when(ci >= 1)
            def _():
                drain(out_cps(ci - 1, nsl))
            issue(in_cps(ci + 1, nsl))

        drain(in_cps(ci, sl))
        compute(sl)
        issue(out_cps(ci, sl))
        return carry

    lax.fori_loop(0, NCHUNK, chunk_step, 0)
    drain(out_cps(NCHUNK - 2, (NCHUNK - 2) % NBUF))
    drain(out_cps(NCHUNK - 1, (NCHUNK - 1) % NBUF))


def kernel(inputs, pos_table):
    mesh = plsc.VectorSubcoreMesh(core_axis_name="c", subcore_axis_name="s")
    return pl.kernel(
        _sc_body,
        mesh=mesh,
        out_type=jax.ShapeDtypeStruct((B, SEQ, D), jnp.float32),
        scratch_types=[
            pltpu.VMEM((NBUF, CHUNK, D), jnp.float32),
            pltpu.VMEM((NBUF, B, CHUNK, D), jnp.float32),
            pltpu.SemaphoreType.DMA((NBUF,)),
            pltpu.SemaphoreType.DMA((NBUF,)),
        ],
    )(inputs, pos_table)


# speedup vs baseline: 5.0599x; 3.0008x over previous
import jax
import jax.numpy as jnp
from jax import lax
from jax.experimental import pallas as pl
from jax.experimental.pallas import tpu as pltpu
from jax.experimental.pallas import tpu_sc as plsc

B, SEQ, D = 4, 8192, 1024
NC, NS = 2, 16
NW = NC * NS
ROWS_PER_W = SEQ // NW
CHUNK = 8
NCHUNK = ROWS_PER_W // CHUNK
NBUF = 2
LANES = 16
VECS_PER_ROW = D // LANES


def _sc_body(in_hbm, tab_hbm, out_hbm, tab_v, io_v, in_sems, out_sems):
    wid = lax.axis_index("s") * NC + lax.axis_index("c")
    row_base = wid * ROWS_PER_W

    def in_cps(ci, sl):
        r0 = row_base + ci * CHUNK
        return [
            pltpu.make_async_copy(
                tab_hbm.at[pl.ds(r0, CHUNK)], tab_v.at[sl], in_sems.at[sl]),
            pltpu.make_async_copy(
                in_hbm.at[:, pl.ds(r0, CHUNK), :], io_v.at[sl], in_sems.at[sl]),
        ]

    def out_cps(ci, sl):
        r0 = row_base + ci * CHUNK
        return [
            pltpu.make_async_copy(
                io_v.at[sl], out_hbm.at[:, pl.ds(r0, CHUNK), :], out_sems.at[sl]),
        ]

    def issue(cps):
        for c in cps:
            c.start()

    def drain(cps):
        for c in cps:
            c.wait()

    def compute(sl):
        @plsc.parallel_loop(0, CHUNK * VECS_PER_ROW, unroll=4)
        def _vec_body(n):
            r = n // VECS_PER_ROW
            col = (n % VECS_PER_ROW) * LANES
            t = tab_v[sl, r, pl.ds(col, LANES)]
            for b in range(B):
                io_v[sl, b, r, pl.ds(col, LANES)] = (
                    io_v[sl, b, r, pl.ds(col, LANES)] + t)

    issue(in_cps(0, 0))

    def chunk_step(ci, carry):
        sl = lax.rem(ci, NBUF)
        nsl = 1 - sl

        @pl.when(ci + 1 < NCHUNK)
        def _prefetch():
            @pl.when(ci >= 1)
            def _():
                drain(out_cps(ci - 1, nsl))
            issue(in_cps(ci + 1, nsl))

        drain(in_cps(ci, sl))
        compute(sl)
        issue(out_cps(ci, sl))
        return carry

    lax.fori_loop(0, NCHUNK, chunk_step, 0)
    drain(out_cps(NCHUNK - 2, (NCHUNK - 2) % NBUF))
    drain(out_cps(NCHUNK - 1, (NCHUNK - 1) % NBUF))


def kernel(inputs, pos_table):
    mesh = plsc.VectorSubcoreMesh(core_axis_name="c", subcore_axis_name="s")
    return pl.kernel(
        _sc_body,
        mesh=mesh,
        out_type=jax.ShapeDtypeStruct((B, SEQ, D), jnp.float32),
        scratch_types=[
            pltpu.VMEM((NBUF, CHUNK, D), jnp.float32),
            pltpu.VMEM((NBUF, B, CHUNK, D), jnp.float32),
            pltpu.SemaphoreType.DMA((NBUF,)),
            pltpu.SemaphoreType.DMA((NBUF,)),
        ],
    )(inputs, pos_table)


# speedup vs baseline: 5.0923x; 1.0064x over previous
import jax
import jax.numpy as jnp
from jax import lax
from jax.experimental import pallas as pl
from jax.experimental.pallas import tpu as pltpu
from jax.experimental.pallas import tpu_sc as plsc

B, SEQ, D = 4, 8192, 1024
NC, NS = 2, 16
NW = NC * NS
ROWS_PER_W = SEQ // NW
CHUNK = 8
NCHUNK = ROWS_PER_W // CHUNK
NBUF = 2
LANES = 16
VECS_PER_ROW = D // LANES


def _sc_body(in_hbm, tab_hbm, out_hbm, tab_v, io_v, in_sems, out_sems):
    wid = lax.axis_index("s") * NC + lax.axis_index("c")
    row_base = wid * ROWS_PER_W

    def in_cps(ci, sl):
        r0 = row_base + ci * CHUNK
        return [
            pltpu.make_async_copy(
                tab_hbm.at[pl.ds(r0, CHUNK)], tab_v.at[sl], in_sems.at[sl]),
            pltpu.make_async_copy(
                in_hbm.at[:, pl.ds(r0, CHUNK), :], io_v.at[sl], in_sems.at[sl]),
        ]

    def out_cps(ci, sl):
        r0 = row_base + ci * CHUNK
        return [
            pltpu.make_async_copy(
                io_v.at[sl], out_hbm.at[:, pl.ds(r0, CHUNK), :], out_sems.at[sl]),
        ]

    def issue(cps):
        for c in cps:
            c.start()

    def drain(cps):
        for c in cps:
            c.wait()

    def compute(sl):
        @plsc.parallel_loop(0, CHUNK * VECS_PER_ROW, unroll=8)
        def _vec_body(n):
            r = n // VECS_PER_ROW
            col = (n % VECS_PER_ROW) * LANES
            t = tab_v[sl, r, pl.ds(col, LANES)]
            for b in range(B):
                plsc.addupdate(io_v.at[sl, b, r, pl.ds(col, LANES)], t)

    issue(in_cps(0, 0))

    def chunk_step(ci, carry):
        sl = lax.rem(ci, NBUF)
        nsl = 1 - sl

        @pl.when(ci + 1 < NCHUNK)
        def _prefetch():
            @pl.when(ci >= 1)
            def _():
                drain(out_cps(ci - 1, nsl))
            issue(in_cps(ci + 1, nsl))

        drain(in_cps(ci, sl))
        compute(sl)
        issue(out_cps(ci, sl))
        return carry

    lax.fori_loop(0, NCHUNK, chunk_step, 0)
    drain(out_cps(NCHUNK - 2, (NCHUNK - 2) % NBUF))
    drain(out_cps(NCHUNK - 1, (NCHUNK - 1) % NBUF))


def kernel(inputs, pos_table):
    mesh = plsc.VectorSubcoreMesh(core_axis_name="c", subcore_axis_name="s")
    return pl.kernel(
        _sc_body,
        mesh=mesh,
        out_type=jax.ShapeDtypeStruct((B, SEQ, D), jnp.float32),
        scratch_types=[
            pltpu.VMEM((NBUF, CHUNK, D), jnp.float32),
            pltpu.VMEM((NBUF, B, CHUNK, D), jnp.float32),
            pltpu.SemaphoreType.DMA((NBUF,)),
            pltpu.SemaphoreType.DMA((NBUF,)),
        ],
    )(inputs, pos_table)


# speedup vs baseline: 5.1229x; 1.0060x over previous
import jax
import jax.numpy as jnp
from jax import lax
from jax.experimental import pallas as pl
from jax.experimental.pallas import tpu as pltpu
from jax.experimental.pallas import tpu_sc as plsc

B, SEQ, D = 4, 8192, 1024
NC, NS = 2, 16
NW = NC * NS
ROWS_PER_W = SEQ // NW
CHUNK = 8
NCHUNK = ROWS_PER_W // CHUNK
NBUF = 3
PREF = NBUF - 1
LANES = 16
VECS_PER_ROW = D // LANES


def _sc_body(in_hbm, tab_hbm, out_hbm, tab_v, io_v, in_sems, out_sems):
    wid = lax.axis_index("s") * NC + lax.axis_index("c")
    row_base = wid * ROWS_PER_W

    def in_cps(ci, sl):
        r0 = row_base + ci * CHUNK
        return [
            pltpu.make_async_copy(
                tab_hbm.at[pl.ds(r0, CHUNK)], tab_v.at[sl], in_sems.at[sl]),
            pltpu.make_async_copy(
                in_hbm.at[:, pl.ds(r0, CHUNK), :], io_v.at[sl], in_sems.at[sl]),
        ]

    def out_cps(ci, sl):
        r0 = row_base + ci * CHUNK
        return [
            pltpu.make_async_copy(
                io_v.at[sl], out_hbm.at[:, pl.ds(r0, CHUNK), :], out_sems.at[sl]),
        ]

    def issue(cps):
        for c in cps:
            c.start()

    def drain(cps):
        for c in cps:
            c.wait()

    def compute(sl):
        @plsc.parallel_loop(0, CHUNK * VECS_PER_ROW, unroll=8)
        def _vec_body(n):
            r = n // VECS_PER_ROW
            col = (n % VECS_PER_ROW) * LANES
            t = tab_v[sl, r, pl.ds(col, LANES)]
            for b in range(B):
                plsc.addupdate(io_v.at[sl, b, r, pl.ds(col, LANES)], t)

    for p in range(PREF):
        issue(in_cps(p, p))

    def chunk_step(ci, carry):
        sl = lax.rem(ci, NBUF)
        cp = ci + PREF
        psl = lax.rem(cp, NBUF)

        @pl.when(cp < NCHUNK)
        def _prefetch():
            @pl.when(cp >= NBUF)
            def _():
                drain(out_cps(cp - NBUF, psl))
            issue(in_cps(cp, psl))

        drain(in_cps(ci, sl))
        compute(sl)
        issue(out_cps(ci, sl))
        return carry

    lax.fori_loop(0, NCHUNK, chunk_step, 0)
    for k in range(NCHUNK - NBUF, NCHUNK):
        drain(out_cps(k, k % NBUF))


def kernel(inputs, pos_table):
    mesh = plsc.VectorSubcoreMesh(core_axis_name="c", subcore_axis_name="s")
    return pl.kernel(
        _sc_body,
        mesh=mesh,
        out_type=jax.ShapeDtypeStruct((B, SEQ, D), jnp.float32),
        scratch_types=[
            pltpu.VMEM((NBUF, CHUNK, D), jnp.float32),
            pltpu.VMEM((NBUF, B, CHUNK, D), jnp.float32),
            pltpu.SemaphoreType.DMA((NBUF,)),
            pltpu.SemaphoreType.DMA((NBUF,)),
        ],
    )(inputs, pos_table)


# speedup vs baseline: 5.2034x; 1.0157x over previous
import jax
import jax.numpy as jnp
from jax import lax
from jax.experimental import pallas as pl
from jax.experimental.pallas import tpu as pltpu
from jax.experimental.pallas import tpu_sc as plsc

B, SEQ, D = 4, 8192, 1024
NC, NS = 2, 16
NW = NC * NS
ROWS_PER_W = SEQ // NW
CHUNK = 4
NCHUNK = ROWS_PER_W // CHUNK
NBUF = 6
PREF = NBUF - 1
LANES = 16
VECS_PER_ROW = D // LANES


def _sc_body(in_hbm, tab_hbm, out_hbm, tab_v, io_v, in_sems, out_sems):
    wid = lax.axis_index("s") * NC + lax.axis_index("c")
    row_base = wid * ROWS_PER_W

    def in_cps(ci, sl):
        r0 = row_base + ci * CHUNK
        return [
            pltpu.make_async_copy(
                tab_hbm.at[pl.ds(r0, CHUNK)], tab_v.at[sl], in_sems.at[sl]),
            pltpu.make_async_copy(
                in_hbm.at[:, pl.ds(r0, CHUNK), :], io_v.at[sl], in_sems.at[sl]),
        ]

    def out_cps(ci, sl):
        r0 = row_base + ci * CHUNK
        return [
            pltpu.make_async_copy(
                io_v.at[sl], out_hbm.at[:, pl.ds(r0, CHUNK), :], out_sems.at[sl]),
        ]

    def issue(cps):
        for c in cps:
            c.start()

    def drain(cps):
        for c in cps:
            c.wait()

    def compute(sl):
        @plsc.parallel_loop(0, CHUNK * VECS_PER_ROW, unroll=8)
        def _vec_body(n):
            r = n // VECS_PER_ROW
            col = (n % VECS_PER_ROW) * LANES
            t = tab_v[sl, r, pl.ds(col, LANES)]
            for b in range(B):
                plsc.addupdate(io_v.at[sl, b, r, pl.ds(col, LANES)], t)

    for p in range(PREF):
        issue(in_cps(p, p))

    def chunk_step(ci, carry):
        sl = lax.rem(ci, NBUF)
        cp = ci + PREF
        psl = lax.rem(cp, NBUF)

        @pl.when(cp < NCHUNK)
        def _prefetch():
            @pl.when(cp >= NBUF)
            def _():
                drain(out_cps(cp - NBUF, psl))
            issue(in_cps(cp, psl))

        drain(in_cps(ci, sl))
        compute(sl)
        issue(out_cps(ci, sl))
        return carry

    lax.fori_loop(0, NCHUNK, chunk_step, 0)
    for k in range(NCHUNK - NBUF, NCHUNK):
        drain(out_cps(k, k % NBUF))


def kernel(inputs, pos_table):
    mesh = plsc.VectorSubcoreMesh(core_axis_name="c", subcore_axis_name="s")
    return pl.kernel(
        _sc_body,
        mesh=mesh,
        out_type=jax.ShapeDtypeStruct((B, SEQ, D), jnp.float32),
        scratch_types=[
            pltpu.VMEM((NBUF, CHUNK, D), jnp.float32),
            pltpu.VMEM((NBUF, B, CHUNK, D), jnp.float32),
            pltpu.SemaphoreType.DMA((NBUF,)),
            pltpu.SemaphoreType.DMA((NBUF,)),
        ],
    )(inputs, pos_table)
